# baseline (speedup 1.0000x reference)
import functools

import jax
import jax.numpy as jnp
from jax import lax
from jax.experimental import pallas as pl
from jax.experimental.pallas import tpu as pltpu
from jax.experimental.pallas import tpu_sc as plsc

TOKEN_TILE = 4096
N_CORES = 2
N_SUBCORES = 16
N_WORKERS = N_CORES * N_SUBCORES


def _logits_kernel(x_ref, wt_ref, b_ref, lt_ref):
    x = x_ref[...]
    wt = wt_ref[...]
    b = b_ref[...]
    nl = jax.lax.dot_general(
        x, wt, (((1,), (0,)), ((), ())), preferred_element_type=jnp.float32
    )
    lt_ref[...] = nl.T + b


def _make_sc_topk(n_tokens, n_exp):
    tpw = n_tokens // N_WORKERS
    mesh = plsc.VectorSubcoreMesh(core_axis_name="c", subcore_axis_name="s")

    @functools.partial(
        pl.kernel,
        mesh=mesh,
        out_type=[
            jax.ShapeDtypeStruct((n_tokens * n_exp,), jnp.float32),
            jax.ShapeDtypeStruct((n_tokens * 2,), jnp.int32),
        ],
        scratch_types=[
            pltpu.VMEM((n_exp, tpw), jnp.float32),
            pltpu.VMEM((tpw * n_exp,), jnp.float32),
            pltpu.VMEM((tpw * 2,), jnp.int32),
        ],
        compiler_params=pltpu.CompilerParams(needs_layout_passes=False),
    )
    def sc_topk(lt_hbm, out_hbm, idx_hbm, lv, ov, iv):
        wid = lax.axis_index("s") * N_CORES + lax.axis_index("c")
        base = wid * tpw
        pltpu.sync_copy(lt_hbm.at[:, pl.ds(base, tpw)], lv)

        zeros16 = jnp.zeros((16,), jnp.float32)

        def zbody(z, carry):
            ov[pl.ds(z * 16, 16)] = zeros16
            return carry

        lax.fori_loop(0, tpw * n_exp // 16, zbody, 0)

        iota16 = lax.iota(jnp.int32, 16)

        def gbody(g, carry):
            t0 = g * 16
            t = t0 + iota16
            v1 = lv[0, pl.ds(t0, 16)]
            i1 = jnp.zeros((16,), jnp.int32)
            v2 = jnp.full((16,), -jnp.inf, jnp.float32)
            i2 = jnp.full((16,), n_exp, jnp.int32)
            for e in range(1, n_exp):
                ve = lv[e, pl.ds(t0, 16)]
                ec = jnp.full((16,), e, jnp.int32)
                gt = ve > v1
                cv = jnp.where(gt, v1, ve)
                ci = jnp.where(gt, i1, ec)
                gt2 = cv > v2
                v2 = jnp.where(gt2, cv, v2)
                i2 = jnp.where(gt2, ci, i2)
                v1 = jnp.where(gt, ve, v1)
                i1 = jnp.where(gt, ec, i1)
            s = jnp.exp(v2 - v1)
            p1 = 1.0 / (1.0 + s)
            p2 = s * p1
            te = t * n_exp
            plsc.store_scatter(ov, [te + i1], p1)
            plsc.store_scatter(ov, [te + i2], p2)
            t2 = t * 2
            plsc.store_scatter(iv, [t2], i1)
            plsc.store_scatter(iv, [t2 + 1], i2)
            return carry

        lax.fori_loop(0, tpw // 16, gbody, 0)

        pltpu.sync_copy(ov, out_hbm.at[pl.ds(base * n_exp, tpw * n_exp)])
        pltpu.sync_copy(iv, idx_hbm.at[pl.ds(base * 2, tpw * 2)])

    return sc_topk


@jax.jit
def kernel(x, W_route, b_route, W_noise, b_noise):
    n_tokens, d = x.shape
    n_exp = W_noise.shape[0]
    wt = W_noise.T
    b = b_noise.reshape(n_exp, 1)
    t = TOKEN_TILE
    lt = pl.pallas_call(
        _logits_kernel,
        grid=(n_tokens // t,),
        compiler_params=pltpu.CompilerParams(
            dimension_semantics=("parallel",)
        ),
        in_specs=[
            pl.BlockSpec((t, d), lambda i: (i, 0)),
            pl.BlockSpec((d, n_exp), lambda i: (0, 0)),
            pl.BlockSpec((n_exp, 1), lambda i: (0, 0)),
        ],
        out_specs=pl.BlockSpec((n_exp, t), lambda i: (0, i)),
        out_shape=jax.ShapeDtypeStruct((n_exp, n_tokens), jnp.float32),
    )(x, wt, b)
    out_flat, idx_flat = _make_sc_topk(n_tokens, n_exp)(lt)
    return (out_flat.reshape(n_tokens, n_exp), idx_flat.reshape(n_tokens, 2))


# speedup vs baseline: 2.9184x; 2.9184x over previous
import jax
import jax.numpy as jnp
from jax.experimental import pallas as pl
from jax.experimental.pallas import tpu as pltpu

TOKEN_TILE = 4096


def _router_kernel(x_ref, wt_ref, b_ref, out_ref, idx_ref):
    x = x_ref[...]
    wt = wt_ref[...]
    b = b_ref[...]
    nl = jax.lax.dot_general(
        x, wt, (((1,), (0,)), ((), ())), preferred_element_type=jnp.float32
    )
    nlt = nl.T + b
    n_exp = nlt.shape[0]
    subl = jax.lax.broadcasted_iota(jnp.int32, nlt.shape, 0)
    big = jnp.int32(n_exp)
    v1 = jnp.max(nlt, axis=0, keepdims=True)
    i1 = jnp.min(jnp.where(nlt == v1, subl, big), axis=0, keepdims=True)
    masked = jnp.where(subl == i1, -jnp.inf, nlt)
    v2 = jnp.max(masked, axis=0, keepdims=True)
    i2 = jnp.min(jnp.where(masked == v2, subl, big), axis=0, keepdims=True)
    s = jnp.exp(v2 - v1)
    p1 = 1.0 / (1.0 + s)
    p2 = s * p1
    out_ref[...] = jnp.where(subl == i1, p1, 0.0) + jnp.where(subl == i2, p2, 0.0)
    idx_ref[...] = jnp.concatenate([i1, i2], axis=0)


@jax.jit
def kernel(x, W_route, b_route, W_noise, b_noise):
    n_tokens, d = x.shape
    n_exp = W_noise.shape[0]
    wt = W_noise.T
    b = b_noise.reshape(n_exp, 1)
    t = TOKEN_TILE
    out_t, idx_t = pl.pallas_call(
        _router_kernel,
        grid=(n_tokens // t,),
        compiler_params=pltpu.CompilerParams(
            dimension_semantics=("parallel",)
        ),
        in_specs=[
            pl.BlockSpec((t, d), lambda i: (i, 0)),
            pl.BlockSpec((d, n_exp), lambda i: (0, 0)),
            pl.BlockSpec((n_exp, 1), lambda i: (0, 0)),
        ],
        out_specs=[
            pl.BlockSpec((n_exp, t), lambda i: (0, i)),
            pl.BlockSpec((2, t), lambda i: (0, i)),
        ],
        out_shape=[
            jax.ShapeDtypeStruct((n_exp, n_tokens), jnp.float32),
            jax.ShapeDtypeStruct((2, n_tokens), jnp.int32),
        ],
    )(x, wt, b)
    return (out_t.T, idx_t.T)
